# baseline (speedup 1.0000x reference)
import functools

import jax
import jax.numpy as jnp
from jax import lax
from jax.experimental import pallas as pl
from jax.experimental.pallas import tpu as pltpu
from jax.experimental.pallas import tpu_sc as plsc

ROWS = 8192
DIM = 128
HALF = DIM // 2
NC = 2
NS = 16
NW = NC * NS
CHUNK = 128
LANES = 16
NB = HALF // LANES


def _emb_kernel(n_idx, table_hbm, idx_hbm, out_hbm, table_sh, idx_v, h_a,
                h_b, f_a, f_b, g_a, g_b, s_a, s_b):
    cid = lax.axis_index("c")
    sid = lax.axis_index("s")
    wid = cid * NS + sid

    per_w = n_idx // NW
    n_chunk = per_w // CHUNK
    rows_per_tile = ROWS // NS

    pltpu.sync_copy(table_hbm.at[pl.ds(sid * rows_per_tile, rows_per_tile)],
                    table_sh.at[pl.ds(sid * rows_per_tile, rows_per_tile)])
    pltpu.sync_copy(idx_hbm.at[pl.ds(wid * n_chunk, n_chunk)], idx_v)
    plsc.subcore_barrier()

    base = wid * per_w

    def gather(ci, buf, sem):
        return pltpu.async_copy(table_sh.at[idx_v.at[ci]], buf, sem)

    def wait_gather(ci, buf, sem):
        pltpu.make_async_copy(table_sh.at[idx_v.at[ci]], buf, sem).wait()

    def store(ci, buf, sem):
        return pltpu.async_copy(
            buf, out_hbm.at[pl.ds((base + ci * CHUNK) * DIM, CHUNK * DIM)],
            sem)

    def wait_store(ci, buf, sem):
        pltpu.make_async_copy(
            buf, out_hbm.at[pl.ds((base + ci * CHUNK) * DIM, CHUNK * DIM)],
            sem).wait()

    def convert(hbuf, fbuf):
        @plsc.parallel_loop(0, CHUNK, unroll=4)
        def _(r):
            off = pl.multiple_of(r * DIM, DIM)
            for c in range(NB):
                w = hbuf[r, pl.ds(LANES * c, LANES)]
                fbuf[pl.ds(off + LANES * c, LANES)] = lax.shift_left(
                    w, jnp.int32(16))
                fbuf[pl.ds(off + HALF + LANES * c, LANES)] = lax.bitwise_and(
                    w, jnp.int32(-65536))

    gather(0, h_a, g_a)
    gather(1, h_b, g_b)
    wait_gather(0, h_a, g_a)
    convert(h_a, f_a)
    store(0, f_a, s_a)
    gather(2, h_a, g_a)
    wait_gather(1, h_b, g_b)
    convert(h_b, f_b)
    store(1, f_b, s_b)
    gather(3, h_b, g_b)

    def body(g, carry):
        c0 = 2 * g
        wait_gather(c0, h_a, g_a)
        wait_store(c0 - 2, f_a, s_a)
        convert(h_a, f_a)
        store(c0, f_a, s_a)
        gather(c0 + 2, h_a, g_a)
        wait_gather(c0 + 1, h_b, g_b)
        wait_store(c0 - 1, f_b, s_b)
        convert(h_b, f_b)
        store(c0 + 1, f_b, s_b)
        gather(c0 + 3, h_b, g_b)
        return carry

    lax.fori_loop(1, n_chunk // 2 - 1, body, 0)

    last = n_chunk - 2
    wait_gather(last, h_a, g_a)
    wait_store(last - 2, f_a, s_a)
    convert(h_a, f_a)
    store(last, f_a, s_a)
    wait_gather(last + 1, h_b, g_b)
    wait_store(last - 1, f_b, s_b)
    convert(h_b, f_b)
    store(last + 1, f_b, s_b)
    wait_store(last, f_a, s_a)
    wait_store(last + 1, f_b, s_b)


@functools.partial(jax.jit, static_argnums=(2,))
def _run(table_pk, idx2d, n_idx):
    mesh = plsc.VectorSubcoreMesh(core_axis_name="c", subcore_axis_name="s")
    k = functools.partial(
        pl.kernel,
        mesh=mesh,
        compiler_params=pltpu.CompilerParams(use_tc_tiling_on_sc=False),
        out_type=jax.ShapeDtypeStruct((n_idx * DIM,), jnp.int32),
        scratch_types=[
            pltpu.VMEM_SHARED((ROWS, HALF), jnp.int32),
            pltpu.VMEM((n_idx // NW // CHUNK, CHUNK), jnp.int32),
            pltpu.VMEM((CHUNK, HALF), jnp.int32),
            pltpu.VMEM((CHUNK, HALF), jnp.int32),
            pltpu.VMEM((CHUNK * DIM,), jnp.int32),
            pltpu.VMEM((CHUNK * DIM,), jnp.int32),
            pltpu.SemaphoreType.DMA,
            pltpu.SemaphoreType.DMA,
            pltpu.SemaphoreType.DMA,
            pltpu.SemaphoreType.DMA,
        ],
    )(functools.partial(_emb_kernel, n_idx))
    return k(table_pk, idx2d)


def kernel(data, table):
    shape = data.shape
    idx = data.reshape(-1).astype(jnp.int32)
    n_idx = idx.shape[0]
    idx2d = idx.reshape(n_idx // CHUNK, CHUNK)
    tb = table.astype(jnp.bfloat16)
    pairs = jnp.stack([tb[:, :HALF], tb[:, HALF:]], axis=-1)
    table_pk = lax.bitcast_convert_type(pairs, jnp.int32)
    out = _run(table_pk, idx2d, n_idx)
    return lax.bitcast_convert_type(out, jnp.float32).reshape(*shape, DIM)


# speedup vs baseline: 1.5270x; 1.5270x over previous
import functools

import jax
import jax.numpy as jnp
from jax import lax
from jax.experimental import pallas as pl
from jax.experimental.pallas import tpu as pltpu
from jax.experimental.pallas import tpu_sc as plsc

ROWS = 8192
DIM = 128
NC = 2
NS = 16
NW = NC * NS
CHUNK = 128


def _emb_kernel(n_idx, table_hbm, idx_hbm, out_hbm, table_sh, idx_v, buf_a,
                buf_b, g_a, g_b, s_a, s_b):
    cid = lax.axis_index("c")
    sid = lax.axis_index("s")
    wid = cid * NS + sid

    per_w = n_idx // NW
    n_chunk = per_w // CHUNK
    rows_per_tile = ROWS // NS

    pltpu.sync_copy(table_hbm.at[pl.ds(sid * rows_per_tile, rows_per_tile)],
                    table_sh.at[pl.ds(sid * rows_per_tile, rows_per_tile)])
    pltpu.sync_copy(idx_hbm.at[pl.ds(wid * (per_w // CHUNK), per_w // CHUNK)],
                    idx_v)
    plsc.subcore_barrier()

    base = wid * per_w

    def gather(ci, buf, sem):
        return pltpu.async_copy(table_sh.at[idx_v.at[ci]], buf, sem)

    def store(ci, buf, sem):
        return pltpu.async_copy(buf, out_hbm.at[pl.ds(base + ci * CHUNK,
                                                      CHUNK)], sem)

    gather(0, buf_a, g_a)
    gather(1, buf_b, g_b)

    def body(g, carry):
        c0 = 2 * g
        pltpu.make_async_copy(table_sh.at[idx_v.at[c0]], buf_a, g_a).wait()
        sa = store(c0, buf_a, s_a)
        pltpu.make_async_copy(table_sh.at[idx_v.at[c0 + 1]], buf_b, g_b).wait()
        sb = store(c0 + 1, buf_b, s_b)
        sa.wait()
        gather(c0 + 2, buf_a, g_a)
        sb.wait()
        gather(c0 + 3, buf_b, g_b)
        return carry

    lax.fori_loop(0, n_chunk // 2 - 1, body, 0)

    last = n_chunk - 2
    pltpu.make_async_copy(table_sh.at[idx_v.at[last]], buf_a, g_a).wait()
    sa = store(last, buf_a, s_a)
    pltpu.make_async_copy(table_sh.at[idx_v.at[last + 1]], buf_b, g_b).wait()
    sb = store(last + 1, buf_b, s_b)
    sa.wait()
    sb.wait()


@functools.partial(jax.jit, static_argnums=(2,))
def _run(table, idx2d, n_idx):
    mesh = plsc.VectorSubcoreMesh(core_axis_name="c", subcore_axis_name="s")
    k = functools.partial(
        pl.kernel,
        mesh=mesh,
        out_type=jax.ShapeDtypeStruct((n_idx, DIM), jnp.float32),
        scratch_types=[
            pltpu.VMEM_SHARED((ROWS, DIM), jnp.float32),
            pltpu.VMEM((n_idx // NW // CHUNK, CHUNK), jnp.int32),
            pltpu.VMEM((CHUNK, DIM), jnp.float32),
            pltpu.VMEM((CHUNK, DIM), jnp.float32),
            pltpu.SemaphoreType.DMA,
            pltpu.SemaphoreType.DMA,
            pltpu.SemaphoreType.DMA,
            pltpu.SemaphoreType.DMA,
        ],
    )(functools.partial(_emb_kernel, n_idx))
    return k(table, idx2d)


def kernel(data, table):
    shape = data.shape
    idx = data.reshape(-1).astype(jnp.int32)
    n_idx = idx.shape[0]
    idx2d = idx.reshape(n_idx // CHUNK, CHUNK)
    out = _run(table, idx2d, n_idx)
    return out.reshape(*shape, DIM)
